# baseline (speedup 1.0000x reference)
import functools
import jax
import jax.numpy as jnp
from jax import lax
from jax.experimental import pallas as pl
from jax.experimental.pallas import tpu as pltpu

N_GRAPHS = 64
RATIO = 0.8
EPS = 1e-5


def _gcn(x, src, dst, emask, W, b):
    h = x @ W
    n = x.shape[0]
    loop = jnp.arange(n, dtype=src.dtype)
    s = jnp.concatenate([src, loop])
    d = jnp.concatenate([dst, loop])
    w = jnp.concatenate([emask.astype(h.dtype), jnp.ones((n,), h.dtype)])
    deg = jnp.zeros((n,), h.dtype).at[d].add(w)
    dis = jnp.where(deg > 0, deg ** -0.5, 0.0)
    norm = dis[s] * dis[d] * w
    out = jnp.zeros((n, h.shape[1]), h.dtype).at[d].add(h[s] * norm[:, None])
    return out + b


def _block(x, src, dst, emask, W0, b0, W1, b1, linW, linb):
    x1 = jax.nn.relu(_gcn(x, src, dst, emask, W0, b0))
    x2 = jax.nn.relu(_gcn(x1, src, dst, emask, W1, b1))
    return jnp.concatenate([x1, x2], axis=1) @ linW + linb


def _topk_pool(x, edge_index, edge_mask, batch, node_mask, p, ratio, num_graphs):
    scr = jnp.tanh((x @ p) / jnp.linalg.norm(p))
    n = x.shape[0]
    counts = jnp.zeros((num_graphs,), batch.dtype).at[batch].add(node_mask.astype(batch.dtype))
    ptr = jnp.concatenate([jnp.zeros((1,), counts.dtype), jnp.cumsum(counts)])
    idx_in = jnp.clip(jnp.arange(n, dtype=counts.dtype) - ptr[batch], 0, n - 1)
    brow = jnp.where(node_mask, batch, num_graphs)
    dense = jnp.full((num_graphs + 1, n), -jnp.inf, scr.dtype).at[brow, idx_in].set(scr)[:num_graphs]
    order = jnp.argsort(-dense, axis=1)
    rnk = jnp.zeros((num_graphs, n), counts.dtype).at[
        jnp.arange(num_graphs)[:, None], order
    ].set(jnp.broadcast_to(jnp.arange(n, dtype=counts.dtype)[None, :], (num_graphs, n)))
    k = jnp.ceil(ratio * counts.astype(jnp.float32)).astype(counts.dtype)
    rank_i = rnk[batch, idx_in]
    kept = node_mask & (rank_i < k[batch])
    koff = jnp.concatenate([jnp.zeros((1,), counts.dtype), jnp.cumsum(k)])
    npos = koff[batch] + rank_i
    tgt = jnp.where(kept, npos, n)
    xn = jnp.zeros((n + 1, x.shape[1]), x.dtype).at[tgt].set(x * scr[:, None])[:n]
    bnew = jnp.zeros((n + 1,), batch.dtype).at[tgt].set(batch)[:n]
    nmask_new = jnp.arange(n, dtype=counts.dtype) < koff[num_graphs]
    s, d = edge_index[0], edge_index[1]
    em = edge_mask & kept[s] & kept[d]
    src_new = jnp.where(em, npos[s], 0).astype(s.dtype)
    dst_new = jnp.where(em, npos[d], 0).astype(d.dtype)
    einew = jnp.stack([src_new, dst_new])
    return xn, einew, bnew, nmask_new, em


def _gadd(x, b, nmask, G):
    seg = jnp.where(nmask, b, G)
    return jnp.zeros((G + 1, x.shape[1]), x.dtype).at[seg].add(x)[:G]


def _gmax(x, b, nmask, G):
    seg = jnp.where(nmask, b, G)
    return jax.ops.segment_max(x, seg, num_segments=G + 1)[:G]


def _mlp_body(h_ref, g_ref, be_ref, w1_ref, b1_ref, w2_ref, b2_ref, o_ref):
    h = h_ref[...]
    h = (h / jnp.sqrt(1.0 + EPS)) * g_ref[...] + be_ref[...]
    h = jnp.maximum(jnp.dot(h, w1_ref[...], preferred_element_type=jnp.float32)
                    + b1_ref[...], 0.0)
    z = jnp.dot(h, w2_ref[...], preferred_element_type=jnp.float32) + b2_ref[...]
    z = z - jnp.max(z, axis=-1, keepdims=True)
    e = jnp.exp(z)
    o_ref[...] = e / jnp.sum(e, axis=-1, keepdims=True)


def _mlp_head(h, bn_gamma, bn_beta, lin1W, lin1b, lin2W, lin2b):
    G = h.shape[0]
    C = lin2W.shape[1]
    w2 = jnp.zeros((lin2W.shape[0], 128), jnp.float32).at[:, :C].set(lin2W)
    b2 = jnp.full((1, 128), -1e30, jnp.float32).at[0, :C].set(lin2b)
    out = pl.pallas_call(
        _mlp_body,
        out_shape=jax.ShapeDtypeStruct((G, 128), jnp.float32),
    )(h, bn_gamma[None, :], bn_beta[None, :], lin1W, lin1b[None, :], w2, b2)
    return out[:, :C]


def kernel(x, edge_index, batch,
           blk0_W0, blk0_b0, blk0_W1, blk0_b1, blk0_linW, blk0_linb,
           blk1_W0, blk1_b0, blk1_W1, blk1_b1, blk1_linW, blk1_linb,
           blk2_W0, blk2_b0, blk2_W1, blk2_b1, blk2_linW, blk2_linb,
           pool1_p, pool2_p, bn_gamma, bn_beta, lin1W, lin1b, lin2W, lin2b):
    P = dict(
        blk0_W0=blk0_W0, blk0_b0=blk0_b0, blk0_W1=blk0_W1, blk0_b1=blk0_b1,
        blk0_linW=blk0_linW, blk0_linb=blk0_linb,
        blk1_W0=blk1_W0, blk1_b0=blk1_b0, blk1_W1=blk1_W1, blk1_b1=blk1_b1,
        blk1_linW=blk1_linW, blk1_linb=blk1_linb,
        blk2_W0=blk2_W0, blk2_b0=blk2_b0, blk2_W1=blk2_W1, blk2_b1=blk2_b1,
        blk2_linW=blk2_linW, blk2_linb=blk2_linb,
        pool1_p=pool1_p, pool2_p=pool2_p,
    )
    G = N_GRAPHS
    n = x.shape[0]
    nmask = jnp.ones((n,), bool)
    emask = jnp.ones((edge_index.shape[1],), bool)
    x = jax.nn.relu(_block(x, edge_index[0], edge_index[1], emask,
                           P["blk0_W0"], P["blk0_b0"], P["blk0_W1"], P["blk0_b1"],
                           P["blk0_linW"], P["blk0_linb"]))
    xs = [_gadd(x, batch, nmask, G), _gmax(x, batch, nmask, G)]
    ei = edge_index
    for b in (1, 2):
        x, ei, batch, nmask, emask = _topk_pool(
            x, ei, emask, batch, nmask, P["pool%d_p" % b], RATIO, G)
        x = jax.nn.relu(_block(x, ei[0], ei[1], emask,
                               P["blk%d_W0" % b], P["blk%d_b0" % b],
                               P["blk%d_W1" % b], P["blk%d_b1" % b],
                               P["blk%d_linW" % b], P["blk%d_linb" % b]))
        xs.extend([_gadd(x, batch, nmask, G), _gmax(x, batch, nmask, G)])
    h = jnp.concatenate(xs, axis=1)
    return _mlp_head(h, bn_gamma, bn_beta, lin1W, lin1b, lin2W, lin2b)


# speedup vs baseline: 1.1248x; 1.1248x over previous
import functools
import jax
import jax.numpy as jnp
from jax import lax
from jax.experimental import pallas as pl
from jax.experimental.pallas import tpu as pltpu

N_GRAPHS = 64
RATIO = 0.8
EPS = 1e-5


def _gcn(x, src, dst, emask, W, b):
    h = x @ W
    n = x.shape[0]
    loop = jnp.arange(n, dtype=src.dtype)
    s = jnp.concatenate([src, loop])
    d = jnp.concatenate([dst, loop])
    w = jnp.concatenate([emask.astype(h.dtype), jnp.ones((n,), h.dtype)])
    deg = jnp.zeros((n,), h.dtype).at[d].add(w)
    dis = jnp.where(deg > 0, deg ** -0.5, 0.0)
    norm = dis[s] * dis[d] * w
    out = jnp.zeros((n, h.shape[1]), h.dtype).at[d].add(h[s] * norm[:, None])
    return out + b


def _block(x, src, dst, emask, W0, b0, W1, b1, linW, linb):
    x1 = jax.nn.relu(_gcn(x, src, dst, emask, W0, b0))
    x2 = jax.nn.relu(_gcn(x1, src, dst, emask, W1, b1))
    return jnp.concatenate([x1, x2], axis=1) @ linW + linb


def _counts_body(b_ref, v_ref, ratio_ref, k_ref):
    Np = b_ref.shape[1]
    G = k_ref.shape[0]
    gi = lax.broadcasted_iota(jnp.int32, (G, 1), 0).astype(jnp.float32)
    acc = jnp.zeros((G, 1), jnp.float32)
    BJ = 2048
    for c in range(Np // BJ):
        bj = b_ref[:, pl.ds(c * BJ, BJ)]
        vj = v_ref[:, pl.ds(c * BJ, BJ)]
        oh = jnp.where(bj == gi, vj, 0.0)
        acc = acc + jnp.sum(oh, axis=1, keepdims=True)
    k_ref[...] = jnp.ceil(ratio_ref[0, 0] * acc)


def _rank_body(sc_ref, bc_ref, vc_ref, sr_ref, br_ref, vr_ref,
               k_ref, koff_ref, tgt_ref, kept_ref, *, n_sentinel):
    BI = sc_ref.shape[0]
    Np = sr_ref.shape[1]
    G = k_ref.shape[1]
    i0 = pl.program_id(0) * BI
    si = sc_ref[...]
    bi = bc_ref[...]
    vi = vc_ref[...]
    ii = i0 + lax.broadcasted_iota(jnp.int32, (BI, 1), 0).astype(jnp.float32)
    acc = jnp.zeros((BI, 1), jnp.float32)
    BJ = 2048
    for c in range(Np // BJ):
        sj = sr_ref[:, pl.ds(c * BJ, BJ)]
        bj = br_ref[:, pl.ds(c * BJ, BJ)]
        vj = vr_ref[:, pl.ds(c * BJ, BJ)]
        jj = c * BJ + lax.broadcasted_iota(jnp.int32, (1, BJ), 1).astype(jnp.float32)
        beats = (sj > si) | ((sj == si) & (jj < ii))
        cmp = (bj == bi) & (vj > 0.0) & beats
        acc = acc + jnp.sum(cmp.astype(jnp.float32), axis=1, keepdims=True)
    ohm = bi == lax.broadcasted_iota(jnp.int32, (1, G), 1).astype(jnp.float32)
    kb = jnp.sum(jnp.where(ohm, k_ref[...], 0.0), axis=1, keepdims=True)
    kob = jnp.sum(jnp.where(ohm, koff_ref[...], 0.0), axis=1, keepdims=True)
    kept = (vi > 0.0) & (acc < kb)
    tgt_ref[...] = jnp.where(kept, kob + acc, float(n_sentinel))
    kept_ref[...] = kept.astype(jnp.float32)


def _topk_pool(x, edge_index, edge_mask, batch, node_mask, p, ratio, num_graphs):
    n, D = x.shape
    G = num_graphs
    Np = ((n + 2047) // 2048) * 2048
    bpad = jnp.zeros((Np, 1), jnp.float32).at[:n, 0].set(batch.astype(jnp.float32))
    vpad = jnp.zeros((Np, 1), jnp.float32).at[:n, 0].set(node_mask.astype(jnp.float32))

    scr = jnp.tanh((x @ p) / jnp.linalg.norm(p))
    scr_col = jnp.zeros((Np, 1), jnp.float32).at[:n, 0].set(scr)

    b_row = bpad.reshape(1, Np)
    v_row = vpad.reshape(1, Np)
    s_row = scr_col.reshape(1, Np)
    ratio_arr = jnp.full((1, 1), ratio, jnp.float32)

    k_col = pl.pallas_call(
        _counts_body,
        out_shape=jax.ShapeDtypeStruct((G, 1), jnp.float32),
    )(b_row, v_row, ratio_arr)
    k_row = k_col.reshape(1, G)
    koff_full = jnp.concatenate([jnp.zeros((1,), jnp.float32),
                                 jnp.cumsum(k_col[:, 0])])
    koff_row = koff_full[:G].reshape(1, G)
    total_kept = koff_full[G].astype(jnp.int32)

    BI = 256
    tgt_col, kept_col = pl.pallas_call(
        functools.partial(_rank_body, n_sentinel=n),
        grid=(Np // BI,),
        in_specs=[pl.BlockSpec((BI, 1), lambda i: (i, 0)),
                  pl.BlockSpec((BI, 1), lambda i: (i, 0)),
                  pl.BlockSpec((BI, 1), lambda i: (i, 0)),
                  pl.BlockSpec((1, Np), lambda i: (0, 0)),
                  pl.BlockSpec((1, Np), lambda i: (0, 0)),
                  pl.BlockSpec((1, Np), lambda i: (0, 0)),
                  pl.BlockSpec((1, G), lambda i: (0, 0)),
                  pl.BlockSpec((1, G), lambda i: (0, 0))],
        out_specs=[pl.BlockSpec((BI, 1), lambda i: (i, 0)),
                   pl.BlockSpec((BI, 1), lambda i: (i, 0))],
        out_shape=[jax.ShapeDtypeStruct((Np, 1), jnp.float32),
                   jax.ShapeDtypeStruct((Np, 1), jnp.float32)],
    )(scr_col, bpad, vpad, s_row, b_row, v_row, k_row, koff_row)

    tgt = tgt_col[:n, 0].astype(jnp.int32)
    kept = kept_col[:n, 0] > 0.0

    xn = jnp.zeros((n + 1, D), x.dtype).at[tgt].set(x * scr[:, None])[:n]
    bnew = jnp.zeros((n + 1,), batch.dtype).at[tgt].set(batch)[:n]
    nmask_new = jnp.arange(n, dtype=jnp.int32) < total_kept
    s, d = edge_index[0], edge_index[1]
    em = edge_mask & kept[s] & kept[d]
    src_new = jnp.where(em, tgt[s], 0).astype(s.dtype)
    dst_new = jnp.where(em, tgt[d], 0).astype(d.dtype)
    einew = jnp.stack([src_new, dst_new])
    return xn, einew, bnew, nmask_new, em


def _gadd(x, b, nmask, G):
    seg = jnp.where(nmask, b, G)
    return jnp.zeros((G + 1, x.shape[1]), x.dtype).at[seg].add(x)[:G]


def _gmax(x, b, nmask, G):
    seg = jnp.where(nmask, b, G)
    return jax.ops.segment_max(x, seg, num_segments=G + 1)[:G]


def _mlp_body(h_ref, g_ref, be_ref, w1_ref, b1_ref, w2_ref, b2_ref, o_ref):
    h = h_ref[...]
    h = (h / jnp.sqrt(1.0 + EPS)) * g_ref[...] + be_ref[...]
    h = jnp.maximum(jnp.dot(h, w1_ref[...], preferred_element_type=jnp.float32)
                    + b1_ref[...], 0.0)
    z = jnp.dot(h, w2_ref[...], preferred_element_type=jnp.float32) + b2_ref[...]
    z = z - jnp.max(z, axis=-1, keepdims=True)
    e = jnp.exp(z)
    o_ref[...] = e / jnp.sum(e, axis=-1, keepdims=True)


def _mlp_head(h, bn_gamma, bn_beta, lin1W, lin1b, lin2W, lin2b):
    G = h.shape[0]
    C = lin2W.shape[1]
    w2 = jnp.zeros((lin2W.shape[0], 128), jnp.float32).at[:, :C].set(lin2W)
    b2 = jnp.full((1, 128), -1e30, jnp.float32).at[0, :C].set(lin2b)
    out = pl.pallas_call(
        _mlp_body,
        out_shape=jax.ShapeDtypeStruct((G, 128), jnp.float32),
    )(h, bn_gamma[None, :], bn_beta[None, :], lin1W, lin1b[None, :], w2, b2)
    return out[:, :C]


def kernel(x, edge_index, batch,
           blk0_W0, blk0_b0, blk0_W1, blk0_b1, blk0_linW, blk0_linb,
           blk1_W0, blk1_b0, blk1_W1, blk1_b1, blk1_linW, blk1_linb,
           blk2_W0, blk2_b0, blk2_W1, blk2_b1, blk2_linW, blk2_linb,
           pool1_p, pool2_p, bn_gamma, bn_beta, lin1W, lin1b, lin2W, lin2b):
    P = dict(
        blk0_W0=blk0_W0, blk0_b0=blk0_b0, blk0_W1=blk0_W1, blk0_b1=blk0_b1,
        blk0_linW=blk0_linW, blk0_linb=blk0_linb,
        blk1_W0=blk1_W0, blk1_b0=blk1_b0, blk1_W1=blk1_W1, blk1_b1=blk1_b1,
        blk1_linW=blk1_linW, blk1_linb=blk1_linb,
        blk2_W0=blk2_W0, blk2_b0=blk2_b0, blk2_W1=blk2_W1, blk2_b1=blk2_b1,
        blk2_linW=blk2_linW, blk2_linb=blk2_linb,
        pool1_p=pool1_p, pool2_p=pool2_p,
    )
    G = N_GRAPHS
    n = x.shape[0]
    nmask = jnp.ones((n,), bool)
    emask = jnp.ones((edge_index.shape[1],), bool)
    x = jax.nn.relu(_block(x, edge_index[0], edge_index[1], emask,
                           P["blk0_W0"], P["blk0_b0"], P["blk0_W1"], P["blk0_b1"],
                           P["blk0_linW"], P["blk0_linb"]))
    xs = [_gadd(x, batch, nmask, G), _gmax(x, batch, nmask, G)]
    ei = edge_index
    for b in (1, 2):
        x, ei, batch, nmask, emask = _topk_pool(
            x, ei, emask, batch, nmask, P["pool%d_p" % b], RATIO, G)
        x = jax.nn.relu(_block(x, ei[0], ei[1], emask,
                               P["blk%d_W0" % b], P["blk%d_b0" % b],
                               P["blk%d_W1" % b], P["blk%d_b1" % b],
                               P["blk%d_linW" % b], P["blk%d_linb" % b]))
        xs.extend([_gadd(x, batch, nmask, G), _gmax(x, batch, nmask, G)])
    h = jnp.concatenate(xs, axis=1)
    return _mlp_head(h, bn_gamma, bn_beta, lin1W, lin1b, lin2W, lin2b)
